# baseline (speedup 1.0000x reference)
import jax
import jax.numpy as jnp
from jax import lax
from jax.experimental import pallas as pl
from jax.experimental.pallas import tpu as pltpu
from jax.experimental.pallas import tpu_sc as plsc

N = 10000
E = 320000
D = 128
H = 64
C = 16
K = 10
ALPHA = 0.1

NC = 2
NS = 16
LANES = 16
CHUNK = 128
KB = 8
CHS = 80
CH = NC * CHS
EPAD = NC * NS * CHS * CHUNK
NPT = 632
NP = NS * NPT

_SC_PARAMS = pltpu.CompilerParams(use_tc_tiling_on_sc=False)
_MESH = plsc.VectorSubcoreMesh(core_axis_name="c", subcore_axis_name="s")


def _mlp_block(x_ref, w1_ref, b1_ref, w2_ref, b2_ref, o_ref):
    h = jnp.dot(x_ref[...], w1_ref[...], preferred_element_type=jnp.float32)
    h = jnp.maximum(h + b1_ref[...], 0.0)
    o_ref[...] = jnp.dot(h, w2_ref[...], preferred_element_type=jnp.float32) + b2_ref[...]


def _mlp(features, W1, b1, W2, b2):
    R = 1000
    return pl.pallas_call(
        _mlp_block,
        grid=(N // R,),
        in_specs=[
            pl.BlockSpec((R, D), lambda i: (i, 0)),
            pl.BlockSpec((D, H), lambda i: (0, 0)),
            pl.BlockSpec((1, H), lambda i: (0, 0)),
            pl.BlockSpec((H, C), lambda i: (0, 0)),
            pl.BlockSpec((1, C), lambda i: (0, 0)),
        ],
        out_specs=pl.BlockSpec((R, C), lambda i: (i, 0)),
        out_shape=jax.ShapeDtypeStruct((N, C), jnp.float32),
    )(features, W1, b1.reshape(1, H), W2, b2.reshape(1, C))


def _deg_body(src_hbm, dst_hbm, deg_hbm, degbuf_sh, idx_v, ones_v, row_v):
    c = lax.axis_index("c")
    s = lax.axis_index("s")
    base = s * NPT

    @pl.when(c == 0)
    def _():
        pltpu.sync_copy(src_hbm.at[s], idx_v)

    @pl.when(c != 0)
    def _():
        pltpu.sync_copy(dst_hbm.at[s], idx_v)

    def zb(i, _):
        row_v[i, :] = jnp.zeros((LANES,), jnp.float32)
        return 0

    lax.fori_loop(0, NPT, zb, 0)

    def ob(i, _):
        ones_v[i, :] = jnp.ones((LANES,), jnp.float32)
        return 0

    lax.fori_loop(0, CHUNK, ob, 0)
    pltpu.sync_copy(row_v, degbuf_sh.at[pl.ds(base, NPT)])
    plsc.subcore_barrier()

    def eb(j, _):
        pltpu.sync_copy(ones_v, degbuf_sh.at[idx_v.at[j]], add=True)
        return 0

    lax.fori_loop(0, CH, eb, 0)
    plsc.subcore_barrier()
    pltpu.sync_copy(degbuf_sh.at[pl.ds(base, NPT)], row_v)
    pltpu.sync_copy(row_v, deg_hbm.at[c].at[pl.ds(base, NPT)])


def _degrees(src2d, dst2d):
    return pl.kernel(
        _deg_body,
        out_type=jax.ShapeDtypeStruct((NC, NP, C), jnp.float32),
        compiler_params=_SC_PARAMS,
        mesh=_MESH,
        scratch_types=[
            pltpu.VMEM_SHARED((NP, C), jnp.float32),
            pltpu.VMEM((CH, CHUNK), jnp.int32),
            pltpu.VMEM((CHUNK, C), jnp.float32),
            pltpu.VMEM((NPT, C), jnp.float32),
        ],
    )(src2d, dst2d)


def _prep_block(deg_ref, h0_ref, g0_ref, ah0_ref, nsrc_ref, ndst_ref):
    dsrc = deg_ref[0]
    ddst = deg_ref[1]
    nsrc = jnp.where(dsrc > 0, lax.rsqrt(jnp.maximum(dsrc, 1.0)), 0.0)
    ndst = jnp.where(ddst > 0, lax.rsqrt(jnp.maximum(ddst, 1.0)), 0.0)
    h0 = h0_ref[...]
    nsrc_ref[...] = nsrc
    ndst_ref[...] = (1.0 - ALPHA) * ndst
    g0_ref[...] = h0 * nsrc
    ah0_ref[...] = ALPHA * h0


def _prep(deg2, h0p):
    return pl.pallas_call(
        _prep_block,
        out_shape=[jax.ShapeDtypeStruct((NP, C), jnp.float32)] * 4,
    )(deg2, h0p)


def _make_step(first_step):
    def body(src_hbm, dst_hbm, p_hbm, ah0_hbm, nsrc_hbm, ndst_hbm,
             out_hbm, g_hbm,
             agg_sh, src_v, dst_v, ah0_v, nsrc_v, ndst_v,
             agg_v, g_v, rows_v, gsem, ssem):
        c = lax.axis_index("c")
        s = lax.axis_index("s")
        base = s * NPT
        icp1 = pltpu.async_copy(src_hbm.at[c].at[s], src_v, gsem)
        icp2 = pltpu.async_copy(dst_hbm.at[c].at[s], dst_v, gsem)
        pltpu.sync_copy(ah0_hbm.at[pl.ds(base, NPT)], ah0_v)
        pltpu.sync_copy(nsrc_hbm.at[pl.ds(base, NPT)], nsrc_v)
        pltpu.sync_copy(ndst_hbm.at[pl.ds(base, NPT)], ndst_v)
        if first_step:
            pltpu.sync_copy(p_hbm.at[0].at[pl.ds(base, NPT)], g_v)
        else:
            pltpu.sync_copy(p_hbm.at[0].at[pl.ds(base, NPT)], g_v)
            pltpu.sync_copy(p_hbm.at[1].at[pl.ds(base, NPT)], agg_v)

            def ug(i, __):
                hrow = ndst_v[i, :] * (g_v[i, :] + agg_v[i, :]) + ah0_v[i, :]
                g_v[i, :] = hrow * nsrc_v[i, :]
                return 0

            lax.fori_loop(0, NPT, ug, 0, unroll=4)
        pltpu.sync_copy(g_v, g_hbm.at[pl.ds(c * NP + base, NPT)])

        def zb(i, __):
            agg_v[i, :] = jnp.zeros((LANES,), jnp.float32)
            return 0

        lax.fori_loop(0, NPT, zb, 0, unroll=4)
        pltpu.sync_copy(agg_v, agg_sh.at[pl.ds(base, NPT)])
        icp1.wait()
        icp2.wait()
        plsc.subcore_barrier()

        def eb(t, __):
            j0 = t * KB
            gs = [pltpu.async_copy(g_hbm.at[src_v.at[j0 + b]], rows_v.at[b], gsem)
                  for b in range(KB)]
            ss = []
            for b in range(KB):
                gs[b].wait()
                ss.append(pltpu.async_copy(rows_v.at[b], agg_sh.at[dst_v.at[j0 + b]],
                                           ssem, add=True))
            for b in range(KB):
                ss[b].wait()
            return 0

        lax.fori_loop(0, CHS // KB, eb, 0)
        plsc.subcore_barrier()
        pltpu.sync_copy(agg_sh.at[pl.ds(base, NPT)], agg_v)
        pltpu.sync_copy(agg_v, out_hbm.at[c].at[pl.ds(base, NPT)])

    return pl.kernel(
        body,
        out_type=(jax.ShapeDtypeStruct((NC, NP, C), jnp.float32),
                  jax.ShapeDtypeStruct((NC * NP, C), jnp.float32)),
        compiler_params=_SC_PARAMS,
        mesh=_MESH,
        scratch_types=[
            pltpu.VMEM_SHARED((NP, C), jnp.float32),
            pltpu.VMEM((CHS, CHUNK), jnp.int32),
            pltpu.VMEM((CHS, CHUNK), jnp.int32),
            pltpu.VMEM((NPT, C), jnp.float32),
            pltpu.VMEM((NPT, C), jnp.float32),
            pltpu.VMEM((NPT, C), jnp.float32),
            pltpu.VMEM((NPT, C), jnp.float32),
            pltpu.VMEM((NPT, C), jnp.float32),
            pltpu.VMEM((KB, CHUNK, C), jnp.float32),
            pltpu.SemaphoreType.DMA,
            pltpu.SemaphoreType.DMA,
        ],
    )


def _final_block(p_ref, ah0_ref, ndst_ref, o_ref):
    o_ref[...] = ndst_ref[...] * (p_ref[0] + p_ref[1]) + ah0_ref[...]


def _final(p, ah0, ndst):
    return pl.pallas_call(
        _final_block,
        out_shape=jax.ShapeDtypeStruct((NP, C), jnp.float32),
    )(p, ah0, ndst)


def kernel(features, adj, W1, b1, W2, b2):
    h0 = _mlp(features, W1, b1, W2, b2)
    pad = N + (jnp.arange(EPAD - E, dtype=jnp.int32) % (NP - N))
    srcp = jnp.concatenate([adj[0], pad])
    dstp = jnp.concatenate([adj[1], pad])
    deg2 = _degrees(srcp.reshape(NS, CH, CHUNK), dstp.reshape(NS, CH, CHUNK))
    h0p = jnp.pad(h0, ((0, NP - N), (0, 0)))
    g0, ah0, nsrc, ndst = _prep(deg2, h0p)
    src4 = (srcp.reshape(NC, NS, CHS, CHUNK)
            + (jnp.arange(NC, dtype=jnp.int32) * NP)[:, None, None, None])
    dst4 = dstp.reshape(NC, NS, CHS, CHUNK)
    step0 = _make_step(True)
    step = _make_step(False)
    p, _ = step0(src4, dst4, jnp.stack([g0, g0]), ah0, nsrc, ndst)
    for _ in range(K - 1):
        p, _ = step(src4, dst4, p, ah0, nsrc, ndst)
    return _final(p, ah0, ndst)[:N]


# speedup vs baseline: 1.0861x; 1.0861x over previous
import jax
import jax.numpy as jnp
from jax import lax
from jax.experimental import pallas as pl
from jax.experimental.pallas import tpu as pltpu
from jax.experimental.pallas import tpu_sc as plsc

N = 10000
E = 320000
D = 128
H = 64
C = 16
K = 10
ALPHA = 0.1

NC = 2
NS = 16
LANES = 16
CHUNK = 128
KB = 8
CHS = 80
CH = NC * CHS
EPAD = NC * NS * CHS * CHUNK
NPT = 632
NP = NS * NPT

_SC_PARAMS = pltpu.CompilerParams(use_tc_tiling_on_sc=False)
_MESH = plsc.VectorSubcoreMesh(core_axis_name="c", subcore_axis_name="s")


def _mlp_block(x_ref, w1_ref, b1_ref, w2_ref, b2_ref, o_ref):
    h = jnp.dot(x_ref[...], w1_ref[...], preferred_element_type=jnp.float32)
    h = jnp.maximum(h + b1_ref[...], 0.0)
    o_ref[...] = jnp.dot(h, w2_ref[...], preferred_element_type=jnp.float32) + b2_ref[...]


def _mlp(features, W1, b1, W2, b2):
    R = 1000
    return pl.pallas_call(
        _mlp_block,
        grid=(N // R,),
        in_specs=[
            pl.BlockSpec((R, D), lambda i: (i, 0)),
            pl.BlockSpec((D, H), lambda i: (0, 0)),
            pl.BlockSpec((1, H), lambda i: (0, 0)),
            pl.BlockSpec((H, C), lambda i: (0, 0)),
            pl.BlockSpec((1, C), lambda i: (0, 0)),
        ],
        out_specs=pl.BlockSpec((R, C), lambda i: (i, 0)),
        out_shape=jax.ShapeDtypeStruct((N, C), jnp.float32),
    )(features, W1, b1.reshape(1, H), W2, b2.reshape(1, C))


def _deg_body(src_hbm, dst_hbm, deg_hbm, degbuf_sh, idx_v, ones_v, row_v):
    c = lax.axis_index("c")
    s = lax.axis_index("s")
    base = s * NPT

    @pl.when(c == 0)
    def _():
        pltpu.sync_copy(src_hbm.at[s], idx_v)

    @pl.when(c != 0)
    def _():
        pltpu.sync_copy(dst_hbm.at[s], idx_v)

    def zb(i, _):
        row_v[i, :] = jnp.zeros((LANES,), jnp.float32)
        return 0

    lax.fori_loop(0, NPT, zb, 0)

    def ob(i, _):
        ones_v[i, :] = jnp.ones((LANES,), jnp.float32)
        return 0

    lax.fori_loop(0, CHUNK, ob, 0)
    pltpu.sync_copy(row_v, degbuf_sh.at[pl.ds(base, NPT)])
    plsc.subcore_barrier()

    def eb(j, _):
        pltpu.sync_copy(ones_v, degbuf_sh.at[idx_v.at[j]], add=True)
        return 0

    lax.fori_loop(0, CH, eb, 0)
    plsc.subcore_barrier()
    pltpu.sync_copy(degbuf_sh.at[pl.ds(base, NPT)], row_v)
    pltpu.sync_copy(row_v, deg_hbm.at[c].at[pl.ds(base, NPT)])


def _degrees(src2d, dst2d):
    return pl.kernel(
        _deg_body,
        out_type=jax.ShapeDtypeStruct((NC, NP, C), jnp.float32),
        compiler_params=_SC_PARAMS,
        mesh=_MESH,
        scratch_types=[
            pltpu.VMEM_SHARED((NP, C), jnp.float32),
            pltpu.VMEM((CH, CHUNK), jnp.int32),
            pltpu.VMEM((CHUNK, C), jnp.float32),
            pltpu.VMEM((NPT, C), jnp.float32),
        ],
    )(src2d, dst2d)


def _prep_block(deg_ref, h0_ref, g0_ref, ah0_ref, nsrc_ref, ndst_ref):
    dsrc = deg_ref[0]
    ddst = deg_ref[1]
    nsrc = jnp.where(dsrc > 0, lax.rsqrt(jnp.maximum(dsrc, 1.0)), 0.0)
    ndst = jnp.where(ddst > 0, lax.rsqrt(jnp.maximum(ddst, 1.0)), 0.0)
    h0 = h0_ref[...]
    nsrc_ref[...] = nsrc
    ndst_ref[...] = (1.0 - ALPHA) * ndst
    g0_ref[...] = h0 * nsrc
    ah0_ref[...] = ALPHA * h0


def _prep(deg2, h0p):
    return pl.pallas_call(
        _prep_block,
        out_shape=[jax.ShapeDtypeStruct((NP, C), jnp.float32)] * 4,
    )(deg2, h0p)


def _make_step(first_step):
    def body(src_hbm, dst_hbm, p_hbm, ah0_hbm, nsrc_hbm, ndst_hbm, out_hbm,
             g_sh, agg_sh, src_v, dst_v, ah0_v, nsrc_v, ndst_v,
             agg_v, g_v, rows_v, gsem, ssem):
        c = lax.axis_index("c")
        s = lax.axis_index("s")
        base = s * NPT
        icp1 = pltpu.async_copy(src_hbm.at[c].at[s], src_v, gsem)
        icp2 = pltpu.async_copy(dst_hbm.at[c].at[s], dst_v, gsem)
        pltpu.sync_copy(ah0_hbm.at[pl.ds(base, NPT)], ah0_v)
        pltpu.sync_copy(nsrc_hbm.at[pl.ds(base, NPT)], nsrc_v)
        pltpu.sync_copy(ndst_hbm.at[pl.ds(base, NPT)], ndst_v)
        if first_step:
            pltpu.sync_copy(p_hbm.at[0].at[pl.ds(base, NPT)], g_v)
        else:
            pltpu.sync_copy(p_hbm.at[0].at[pl.ds(base, NPT)], g_v)
            pltpu.sync_copy(p_hbm.at[1].at[pl.ds(base, NPT)], agg_v)

            def ug(i, __):
                hrow = ndst_v[i, :] * (g_v[i, :] + agg_v[i, :]) + ah0_v[i, :]
                g_v[i, :] = hrow * nsrc_v[i, :]
                return 0

            lax.fori_loop(0, NPT, ug, 0, unroll=4)
        pltpu.sync_copy(g_v, g_sh.at[pl.ds(base, NPT)])

        def zb(i, __):
            agg_v[i, :] = jnp.zeros((LANES,), jnp.float32)
            return 0

        lax.fori_loop(0, NPT, zb, 0, unroll=4)
        pltpu.sync_copy(agg_v, agg_sh.at[pl.ds(base, NPT)])
        icp1.wait()
        icp2.wait()
        plsc.subcore_barrier()

        def eb(t, __):
            j0 = t * KB
            gs = [pltpu.async_copy(g_sh.at[src_v.at[j0 + b]], rows_v.at[b], gsem)
                  for b in range(KB)]
            ss = []
            for b in range(KB):
                gs[b].wait()
                ss.append(pltpu.async_copy(rows_v.at[b], agg_sh.at[dst_v.at[j0 + b]],
                                           ssem, add=True))
            for b in range(KB):
                ss[b].wait()
            return 0

        lax.fori_loop(0, CHS // KB, eb, 0)
        plsc.subcore_barrier()
        pltpu.sync_copy(agg_sh.at[pl.ds(base, NPT)], agg_v)
        pltpu.sync_copy(agg_v, out_hbm.at[c].at[pl.ds(base, NPT)])

    return pl.kernel(
        body,
        out_type=jax.ShapeDtypeStruct((NC, NP, C), jnp.float32),
        compiler_params=_SC_PARAMS,
        mesh=_MESH,
        scratch_types=[
            pltpu.VMEM_SHARED((NP, C), jnp.float32),
            pltpu.VMEM_SHARED((NP, C), jnp.float32),
            pltpu.VMEM((CHS, CHUNK), jnp.int32),
            pltpu.VMEM((CHS, CHUNK), jnp.int32),
            pltpu.VMEM((NPT, C), jnp.float32),
            pltpu.VMEM((NPT, C), jnp.float32),
            pltpu.VMEM((NPT, C), jnp.float32),
            pltpu.VMEM((NPT, C), jnp.float32),
            pltpu.VMEM((NPT, C), jnp.float32),
            pltpu.VMEM((KB, CHUNK, C), jnp.float32),
            pltpu.SemaphoreType.DMA,
            pltpu.SemaphoreType.DMA,
        ],
    )


def _final_block(p_ref, ah0_ref, ndst_ref, o_ref):
    o_ref[...] = ndst_ref[...] * (p_ref[0] + p_ref[1]) + ah0_ref[...]


def _final(p, ah0, ndst):
    return pl.pallas_call(
        _final_block,
        out_shape=jax.ShapeDtypeStruct((NP, C), jnp.float32),
    )(p, ah0, ndst)


def kernel(features, adj, W1, b1, W2, b2):
    h0 = _mlp(features, W1, b1, W2, b2)
    pad = N + (jnp.arange(EPAD - E, dtype=jnp.int32) % (NP - N))
    srcp = jnp.concatenate([adj[0], pad])
    dstp = jnp.concatenate([adj[1], pad])
    deg2 = _degrees(srcp.reshape(NS, CH, CHUNK), dstp.reshape(NS, CH, CHUNK))
    h0p = jnp.pad(h0, ((0, NP - N), (0, 0)))
    g0, ah0, nsrc, ndst = _prep(deg2, h0p)
    src4 = srcp.reshape(NC, NS, CHS, CHUNK)
    dst4 = dstp.reshape(NC, NS, CHS, CHUNK)
    step0 = _make_step(True)
    step = _make_step(False)
    p = step0(src4, dst4, jnp.stack([g0, g0]), ah0, nsrc, ndst)
    for _ in range(K - 1):
        p = step(src4, dst4, p, ah0, nsrc, ndst)
    return _final(p, ah0, ndst)[:N]
